# baseline (speedup 1.0000x reference)
import jax
import jax.numpy as jnp
from jax.experimental import pallas as pl
from jax.experimental.pallas import tpu as pltpu

_FG = 1
_POS_IOU = 0.2
_ENT_THR = 0.1
_RM_THR = 0.9
_THETA = 0.1

_INTERPRET = False


def _fused_body(x_ref, vmat_ref, pjg_ref, gjg_ref, phi_ref, plo_ref,
                tgt_col_ref, gts_row_ref, gts_col_ref, u_ref,
                iou_out_ref, cls_out_ref,
                maxprob_s, clsl_s, colsum_s):
    s = pl.program_id(0)
    nsteps = pl.num_programs(0)
    cc = x_ref.shape[1]
    m = vmat_ref.shape[0]
    c = vmat_ref.shape[1] - 1
    n_valid = m - (m // 100)

    maxprob_s[pl.ds(s, 1), :] = jnp.max(x_ref[...], axis=(-2, -1))

    base = s * cc
    ones_row = jnp.ones((1, m), jnp.float32)
    ones_col = jnp.ones((cc, 1), jnp.float32)
    lane_f = jax.lax.broadcasted_iota(jnp.int32, (cc, m), 1).astype(jnp.float32)
    pj_f = pjg_ref[pl.ds(base, cc), :].astype(jnp.float32)
    pj_b = jnp.dot(pj_f, ones_row, preferred_element_type=jnp.float32)
    p_onehot = (pj_b == lane_f).astype(jnp.float32)
    g = jnp.dot(p_onehot, vmat_ref[...],
                preferred_element_type=jnp.float32)
    a = g[:, 0:1]
    logits = g[:, 1:]

    gj_f = gjg_ref[pl.ds(base, cc), :].astype(jnp.float32)
    gj_b = jnp.dot(gj_f, ones_row, preferred_element_type=jnp.float32)
    t_onehot = (gj_b == lane_f).astype(jnp.float32)
    tid = jnp.dot(t_onehot, tgt_col_ref[...],
                  preferred_element_type=jnp.float32)
    cls = tid.astype(jnp.int32)

    mx = jnp.max(logits, axis=1, keepdims=True)
    lse = mx + jnp.log(jnp.sum(jnp.exp(logits - mx), axis=1, keepdims=True))
    lane_c = jax.lax.broadcasted_iota(jnp.int32, (cc, c), 1)
    picked = jnp.sum(jnp.where(lane_c == cls, logits, 0.0), axis=1,
                     keepdims=True)
    clsl_s[pl.ds(base, cc), :] = lse - picked

    a_b = jnp.dot(a, ones_row, preferred_element_type=jnp.float32)
    g_b = jnp.dot(ones_col, gts_row_ref[...],
                  preferred_element_type=jnp.float32)
    d = jnp.abs(a_b - g_b)
    f = jnp.where(d < _THETA, d * d * (1.0 / (2.0 * _THETA)), d - 0.5 * _THETA)
    lmask = ((jax.lax.broadcasted_iota(jnp.int32, (1, cc), 1) + base)
             < n_valid).astype(jnp.float32)
    contrib = jnp.dot(lmask, f, preferred_element_type=jnp.float32)

    @pl.when(s == 0)
    def _():
        colsum_s[...] = contrib

    @pl.when(s > 0)
    def _():
        colsum_s[...] += contrib

    @pl.when(s == nsteps - 1)
    def _():
        mat = maxprob_s[...]
        lane_s = jax.lax.broadcasted_iota(jnp.int32, (m, nsteps), 1)
        row_sel = (phi_ref[...] == lane_s).astype(jnp.float32)
        t1 = jnp.dot(row_sel, mat, preferred_element_type=jnp.float32)
        lane_cc = jax.lax.broadcasted_iota(jnp.int32, (m, cc), 1)
        mp = jnp.sum(jnp.where(plo_ref[...] == lane_cc, t1, 0.0), axis=1,
                     keepdims=True)
        removed = (mp < _ENT_THR) & (u_ref[...] < _RM_THR)
        w = jnp.where(removed, 0.0,
                      jnp.where(gts_col_ref[...] < _POS_IOU, 1.0, 2.0))
        valid = jax.lax.broadcasted_iota(jnp.int32, (m, 1), 0) < n_valid
        w = jnp.where(valid, w, 0.0)
        wsum = jnp.sum(w) + 0.0001
        cls_loss = jnp.sum(clsl_s[...] * w) / wsum
        iou_num = jnp.dot(colsum_s[...], w,
                          preferred_element_type=jnp.float32)
        iou_out_ref[...] = iou_num / wsum
        cls_out_ref[...] = jnp.reshape(cls_loss, (1, 1))


@jax.jit
def kernel(cls_logits, iou_scores, map_ious, pred_mask_prob, target_ids,
           map_indices):
    bs, ch, c = cls_logits.shape
    ht, wd = pred_mask_prob.shape[2], pred_mask_prob.shape[3]
    rows = bs * ch
    cc = 20
    nc = ch // cc
    nsteps = bs * nc

    k = ch - _FG
    pad = rows - bs * k
    zpad_i = jnp.zeros((pad,), jnp.int32)
    zpad_f = jnp.zeros((pad,), jnp.float32)

    pj = map_indices[:, 0, _FG:].astype(jnp.int32)
    gj = map_indices[:, 1, _FG:].astype(jnp.int32)
    off = (jnp.arange(bs, dtype=jnp.int32) * ch)[:, None]
    pjg = jnp.concatenate([(pj + off).reshape(-1), zpad_i])
    gjg = jnp.concatenate([(gj + off).reshape(-1), zpad_i])
    phi = (pjg // cc)[:, None]
    plo = (pjg % cc)[:, None]
    vmat = jnp.concatenate(
        [iou_scores.reshape(rows, 1), cls_logits.reshape(rows, c)], axis=1)
    tgt_col = target_ids.astype(jnp.float32).reshape(rows, 1)
    iou = map_ious[:, _FG:].astype(jnp.float32).reshape(-1)
    iou_p = jnp.concatenate([iou, zpad_f])
    gts_row = iou_p.reshape(1, rows)
    gts_col = iou_p.reshape(rows, 1)
    u = jax.random.uniform(jax.random.key(42), (bs, k), dtype=jnp.float32)
    u_col = jnp.concatenate([u.reshape(-1), zpad_f]).reshape(rows, 1)

    full = lambda shape: pl.BlockSpec(shape, lambda s: (0,) * len(shape))
    iou_loss, cls_loss = pl.pallas_call(
        _fused_body,
        grid=(nsteps,),
        in_specs=[
            pl.BlockSpec((1, cc, ht, wd), lambda s: (s // nc, s % nc, 0, 0)),
            full((rows, 1 + c)),
            full((rows, 1)),
            full((rows, 1)),
            full((rows, 1)),
            full((rows, 1)),
            full((rows, 1)),
            full((1, rows)),
            full((rows, 1)),
            full((rows, 1)),
        ],
        out_specs=[full((1, 1)), full((1, 1))],
        out_shape=[jax.ShapeDtypeStruct((1, 1), jnp.float32),
                   jax.ShapeDtypeStruct((1, 1), jnp.float32)],
        scratch_shapes=[
            pltpu.VMEM((nsteps, cc), jnp.float32),
            pltpu.VMEM((rows, 1), jnp.float32),
            pltpu.VMEM((1, rows), jnp.float32),
        ],
        interpret=_INTERPRET,
    )(pred_mask_prob, vmat, pjg[:, None], gjg[:, None], phi, plo,
      tgt_col, gts_row, gts_col, u_col)
    return (iou_loss[0, 0], cls_loss[0, 0])


# speedup vs baseline: 1.3618x; 1.3618x over previous
import jax
import jax.numpy as jnp
from jax.experimental import pallas as pl
from jax.experimental.pallas import tpu as pltpu

_FG = 1
_POS_IOU = 0.2
_ENT_THR = 0.1
_RM_THR = 0.9
_THETA = 0.1

_INTERPRET = False


def _split_hl(x):
    hi = x.astype(jnp.bfloat16).astype(jnp.float32)
    return hi, x - hi


def _fused_body(x_ref, vhi_ref, vlo_ref, pjhi_ref, pjlo_ref, gjhi_ref,
                gjlo_ref, phi_ref, plo_ref, tgt_col_ref, gts_bmat_ref,
                gts_col_ref, u_ref,
                iou_out_ref, cls_out_ref,
                maxprob_s, clsl_s, colsum_s):
    s = pl.program_id(0)
    nsteps = pl.num_programs(0)
    cc = x_ref.shape[1]
    m = vhi_ref.shape[0]
    c = vhi_ref.shape[1] - 1
    n_valid = m - (m // 100)

    def dot(x, y):
        return jax.lax.dot_general(x, y, (((1,), (0,)), ((), ())),
                                   preferred_element_type=jnp.float32)

    maxprob_s[pl.ds(s, 1), :] = jnp.max(x_ref[...], axis=(-2, -1))

    base = s * cc
    ones_row = jnp.ones((1, m), jnp.float32)
    lane_f = jax.lax.broadcasted_iota(jnp.int32, (cc, m), 1).astype(jnp.float32)
    pj_b = (dot(pjhi_ref[pl.ds(base, cc), :], ones_row) +
            dot(pjlo_ref[pl.ds(base, cc), :], ones_row))
    p_onehot = (pj_b == lane_f).astype(jnp.float32)
    g = dot(p_onehot, vhi_ref[...]) + dot(p_onehot, vlo_ref[...])
    a = g[:, 0:1]
    logits = g[:, 1:]

    gj_b = (dot(gjhi_ref[pl.ds(base, cc), :], ones_row) +
            dot(gjlo_ref[pl.ds(base, cc), :], ones_row))
    t_onehot = (gj_b == lane_f).astype(jnp.float32)
    tid = dot(t_onehot, tgt_col_ref[...])
    cls = tid.astype(jnp.int32)

    mx = jnp.max(logits, axis=1, keepdims=True)
    lse = mx + jnp.log(jnp.sum(jnp.exp(logits - mx), axis=1, keepdims=True))
    lane_c = jax.lax.broadcasted_iota(jnp.int32, (cc, c), 1)
    picked = jnp.sum(jnp.where(lane_c == cls, logits, 0.0), axis=1,
                     keepdims=True)
    clsl_s[pl.ds(base, cc), :] = lse - picked

    a_hi, a_lo = _split_hl(a)
    a_b = dot(a_hi, ones_row) + dot(a_lo, ones_row)
    d = jnp.abs(a_b - gts_bmat_ref[...])
    f = jnp.where(d < _THETA, d * d * (1.0 / (2.0 * _THETA)), d - 0.5 * _THETA)
    lmask = ((jax.lax.broadcasted_iota(jnp.int32, (1, cc), 1) + base)
             < n_valid).astype(jnp.float32)
    f_hi, f_lo = _split_hl(f)
    contrib = dot(lmask, f_hi) + dot(lmask, f_lo)

    @pl.when(s == 0)
    def _():
        colsum_s[...] = contrib

    @pl.when(s > 0)
    def _():
        colsum_s[...] += contrib

    @pl.when(s == nsteps - 1)
    def _():
        mat = maxprob_s[...]
        mat_hi, mat_lo = _split_hl(mat)
        lane_s = jax.lax.broadcasted_iota(jnp.int32, (m, nsteps), 1)
        row_sel = (phi_ref[...] == lane_s).astype(jnp.float32)
        t1 = dot(row_sel, mat_hi) + dot(row_sel, mat_lo)
        lane_cc = jax.lax.broadcasted_iota(jnp.int32, (m, cc), 1)
        mp = jnp.sum(jnp.where(plo_ref[...] == lane_cc, t1, 0.0), axis=1,
                     keepdims=True)
        removed = (mp < _ENT_THR) & (u_ref[...] < _RM_THR)
        w = jnp.where(removed, 0.0,
                      jnp.where(gts_col_ref[...] < _POS_IOU, 1.0, 2.0))
        valid = jax.lax.broadcasted_iota(jnp.int32, (m, 1), 0) < n_valid
        w = jnp.where(valid, w, 0.0)
        wsum = jnp.sum(w) + 0.0001
        cls_loss = jnp.sum(clsl_s[...] * w) / wsum
        cs_hi, cs_lo = _split_hl(colsum_s[...])
        iou_num = dot(cs_hi, w) + dot(cs_lo, w)
        iou_out_ref[...] = iou_num / wsum
        cls_out_ref[...] = jnp.reshape(cls_loss, (1, 1))


@jax.jit
def kernel(cls_logits, iou_scores, map_ious, pred_mask_prob, target_ids,
           map_indices):
    bs, ch, c = cls_logits.shape
    ht, wd = pred_mask_prob.shape[2], pred_mask_prob.shape[3]
    rows = bs * ch
    cc = 50
    nc = ch // cc
    nsteps = bs * nc

    k = ch - _FG
    pad = rows - bs * k
    zpad_i = jnp.zeros((pad,), jnp.int32)
    zpad_f = jnp.zeros((pad,), jnp.float32)

    pj = map_indices[:, 0, _FG:].astype(jnp.int32)
    gj = map_indices[:, 1, _FG:].astype(jnp.int32)
    off = (jnp.arange(bs, dtype=jnp.int32) * ch)[:, None]
    pjg = jnp.concatenate([(pj + off).reshape(-1), zpad_i])
    gjg = jnp.concatenate([(gj + off).reshape(-1), zpad_i])

    def _hl_col(x_int):
        xf = x_int.astype(jnp.float32)[:, None]
        hi = xf.astype(jnp.bfloat16).astype(jnp.float32)
        return hi, xf - hi

    pjhi, pjlo = _hl_col(pjg)
    gjhi, gjlo = _hl_col(gjg)
    phi = (pjg // cc)[:, None]
    plo = (pjg % cc)[:, None]
    vmat = jnp.concatenate(
        [iou_scores.reshape(rows, 1), cls_logits.reshape(rows, c)], axis=1)
    vhi = vmat.astype(jnp.bfloat16).astype(jnp.float32)
    vlo = vmat - vhi
    tgt_col = target_ids.astype(jnp.float32).reshape(rows, 1)
    iou = map_ious[:, _FG:].astype(jnp.float32).reshape(-1)
    iou_p = jnp.concatenate([iou, zpad_f])
    gts_bmat = jnp.broadcast_to(iou_p.reshape(1, rows), (cc, rows))
    gts_col = iou_p.reshape(rows, 1)
    u = jax.random.uniform(jax.random.key(42), (bs, k), dtype=jnp.float32)
    u_col = jnp.concatenate([u.reshape(-1), zpad_f]).reshape(rows, 1)

    full = lambda shape: pl.BlockSpec(shape, lambda s: (0,) * len(shape))
    iou_loss, cls_loss = pl.pallas_call(
        _fused_body,
        grid=(nsteps,),
        in_specs=[
            pl.BlockSpec((1, cc, ht, wd), lambda s: (s // nc, s % nc, 0, 0)),
            full((rows, 1 + c)),
            full((rows, 1 + c)),
            full((rows, 1)),
            full((rows, 1)),
            full((rows, 1)),
            full((rows, 1)),
            full((rows, 1)),
            full((rows, 1)),
            full((rows, 1)),
            full((cc, rows)),
            full((rows, 1)),
            full((rows, 1)),
        ],
        out_specs=[full((1, 1)), full((1, 1))],
        out_shape=[jax.ShapeDtypeStruct((1, 1), jnp.float32),
                   jax.ShapeDtypeStruct((1, 1), jnp.float32)],
        scratch_shapes=[
            pltpu.VMEM((nsteps, cc), jnp.float32),
            pltpu.VMEM((rows, 1), jnp.float32),
            pltpu.VMEM((1, rows), jnp.float32),
        ],
        interpret=_INTERPRET,
    )(pred_mask_prob, vhi, vlo, pjhi, pjlo, gjhi, gjlo, phi, plo,
      tgt_col, gts_bmat, gts_col, u_col)
    return (iou_loss[0, 0], cls_loss[0, 0])


# speedup vs baseline: 1.3900x; 1.0207x over previous
import jax
import jax.numpy as jnp
from jax.experimental import pallas as pl
from jax.experimental.pallas import tpu as pltpu

_FG = 1
_POS_IOU = 0.2
_ENT_THR = 0.1
_RM_THR = 0.9
_THETA = 0.1

_INTERPRET = False


def _split_hl(x):
    hi = x.astype(jnp.bfloat16).astype(jnp.float32)
    return hi, x - hi


def _fused_body(x_ref, vhi_ref, vlo_ref, pjhi_ref, pjlo_ref, gjhi_ref,
                gjlo_ref, phi_ref, plo_ref, tgt_col_ref, gts_bmat_ref,
                gts_col_ref, u_ref,
                iou_out_ref, cls_out_ref,
                maxprob_s, clsl_s, colsum_s):
    s = pl.program_id(0)
    nsteps = pl.num_programs(0)
    cc = x_ref.shape[1]
    m = vhi_ref.shape[0]
    c = vhi_ref.shape[1] - 1
    n_valid = m - (m // 100)

    def dot(x, y):
        return jax.lax.dot_general(x, y, (((1,), (0,)), ((), ())),
                                   preferred_element_type=jnp.float32)

    maxprob_s[pl.ds(s, 1), :] = jnp.max(x_ref[...], axis=(-2, -1))

    base = s * cc
    ones_row = jnp.ones((1, m), jnp.float32)
    lane_f = jax.lax.broadcasted_iota(jnp.int32, (cc, m), 1).astype(jnp.float32)
    pj_b = (dot(pjhi_ref[pl.ds(base, cc), :], ones_row) +
            dot(pjlo_ref[pl.ds(base, cc), :], ones_row))
    p_onehot = (pj_b == lane_f).astype(jnp.float32)
    g = dot(p_onehot, vhi_ref[...]) + dot(p_onehot, vlo_ref[...])
    a = g[:, 0:1]
    logits = g[:, 1:]

    gj_b = (dot(gjhi_ref[pl.ds(base, cc), :], ones_row) +
            dot(gjlo_ref[pl.ds(base, cc), :], ones_row))
    t_onehot = (gj_b == lane_f).astype(jnp.float32)
    tid = dot(t_onehot, tgt_col_ref[...])
    cls = tid.astype(jnp.int32)

    mx = jnp.max(logits, axis=1, keepdims=True)
    lse = mx + jnp.log(jnp.sum(jnp.exp(logits - mx), axis=1, keepdims=True))
    lane_c = jax.lax.broadcasted_iota(jnp.int32, (cc, c), 1)
    picked = jnp.sum(jnp.where(lane_c == cls, logits, 0.0), axis=1,
                     keepdims=True)
    clsl_s[pl.ds(base, cc), :] = lse - picked

    a_hi, a_lo = _split_hl(a)
    a_b = dot(a_hi, ones_row) + dot(a_lo, ones_row)
    d = jnp.abs(a_b - gts_bmat_ref[...])
    f = jnp.where(d < _THETA, d * d * (1.0 / (2.0 * _THETA)), d - 0.5 * _THETA)
    lmask = ((jax.lax.broadcasted_iota(jnp.int32, (1, cc), 1) + base)
             < n_valid).astype(jnp.float32)
    f_hi, f_lo = _split_hl(f)
    contrib = dot(lmask, f_hi) + dot(lmask, f_lo)

    @pl.when(s == 0)
    def _():
        colsum_s[...] = contrib

    @pl.when(s > 0)
    def _():
        colsum_s[...] += contrib

    @pl.when(s == nsteps - 1)
    def _():
        mat = maxprob_s[...]
        mat_hi, mat_lo = _split_hl(mat)
        lane_s = jax.lax.broadcasted_iota(jnp.int32, (m, nsteps), 1)
        row_sel = (phi_ref[...] == lane_s).astype(jnp.float32)
        t1 = dot(row_sel, mat_hi) + dot(row_sel, mat_lo)
        lane_cc = jax.lax.broadcasted_iota(jnp.int32, (m, cc), 1)
        mp = jnp.sum(jnp.where(plo_ref[...] == lane_cc, t1, 0.0), axis=1,
                     keepdims=True)
        removed = (mp < _ENT_THR) & (u_ref[...] < _RM_THR)
        w = jnp.where(removed, 0.0,
                      jnp.where(gts_col_ref[...] < _POS_IOU, 1.0, 2.0))
        valid = jax.lax.broadcasted_iota(jnp.int32, (m, 1), 0) < n_valid
        w = jnp.where(valid, w, 0.0)
        wsum = jnp.sum(w) + 0.0001
        cls_loss = jnp.sum(clsl_s[...] * w) / wsum
        cs_hi, cs_lo = _split_hl(colsum_s[...])
        iou_num = dot(cs_hi, w) + dot(cs_lo, w)
        iou_out_ref[...] = iou_num / wsum
        cls_out_ref[...] = jnp.reshape(cls_loss, (1, 1))


@jax.jit
def kernel(cls_logits, iou_scores, map_ious, pred_mask_prob, target_ids,
           map_indices):
    bs, ch, c = cls_logits.shape
    ht, wd = pred_mask_prob.shape[2], pred_mask_prob.shape[3]
    rows = bs * ch
    cc = 50
    nc = ch // cc
    nsteps = bs * nc

    k = ch - _FG
    pad = rows - bs * k
    zpad_i = jnp.zeros((pad,), jnp.int32)
    zpad_f = jnp.zeros((pad,), jnp.float32)

    pj = map_indices[:, 0, _FG:].astype(jnp.int32)
    gj = map_indices[:, 1, _FG:].astype(jnp.int32)
    off = (jnp.arange(bs, dtype=jnp.int32) * ch)[:, None]
    pjg = jnp.concatenate([(pj + off).reshape(-1), zpad_i])
    gjg = jnp.concatenate([(gj + off).reshape(-1), zpad_i])

    def _hl_col(x_int):
        xf = x_int.astype(jnp.float32)[:, None]
        hi = jax.lax.optimization_barrier(
            xf.astype(jnp.bfloat16)).astype(jnp.float32)
        return hi, xf - hi

    pjhi, pjlo = _hl_col(pjg)
    gjhi, gjlo = _hl_col(gjg)
    phi = (pjg // cc)[:, None]
    plo = (pjg % cc)[:, None]
    vmat = jnp.concatenate(
        [iou_scores.reshape(rows, 1), cls_logits.reshape(rows, c)], axis=1)
    vhi = jax.lax.optimization_barrier(
        vmat.astype(jnp.bfloat16)).astype(jnp.float32)
    vlo = vmat - vhi
    tgt_col = target_ids.astype(jnp.float32).reshape(rows, 1)
    iou = map_ious[:, _FG:].astype(jnp.float32).reshape(-1)
    iou_p = jnp.concatenate([iou, zpad_f])
    gts_bmat = jnp.broadcast_to(iou_p.reshape(1, rows), (cc, rows))
    gts_col = iou_p.reshape(rows, 1)
    u = jax.random.uniform(jax.random.key(42), (bs, k), dtype=jnp.float32)
    u_col = jnp.concatenate([u.reshape(-1), zpad_f]).reshape(rows, 1)

    full = lambda shape: pl.BlockSpec(shape, lambda s: (0,) * len(shape))
    iou_loss, cls_loss = pl.pallas_call(
        _fused_body,
        grid=(nsteps,),
        in_specs=[
            pl.BlockSpec((1, cc, ht, wd), lambda s: (s // nc, s % nc, 0, 0)),
            full((rows, 1 + c)),
            full((rows, 1 + c)),
            full((rows, 1)),
            full((rows, 1)),
            full((rows, 1)),
            full((rows, 1)),
            full((rows, 1)),
            full((rows, 1)),
            full((rows, 1)),
            full((cc, rows)),
            full((rows, 1)),
            full((rows, 1)),
        ],
        out_specs=[full((1, 1)), full((1, 1))],
        out_shape=[jax.ShapeDtypeStruct((1, 1), jnp.float32),
                   jax.ShapeDtypeStruct((1, 1), jnp.float32)],
        scratch_shapes=[
            pltpu.VMEM((nsteps, cc), jnp.float32),
            pltpu.VMEM((rows, 1), jnp.float32),
            pltpu.VMEM((1, rows), jnp.float32),
        ],
        interpret=_INTERPRET,
    )(pred_mask_prob, vhi, vlo, pjhi, pjlo, gjhi, gjlo, phi, plo,
      tgt_col, gts_bmat, gts_col, u_col)
    return (iou_loss[0, 0], cls_loss[0, 0])


# speedup vs baseline: 1.6451x; 1.1835x over previous
import jax
import jax.numpy as jnp
from jax.experimental import pallas as pl
from jax.experimental.pallas import tpu as pltpu

_FG = 1
_POS_IOU = 0.2
_ENT_THR = 0.1
_RM_THR = 0.9
_THETA = 0.1

_INTERPRET = False


def _split_hl(x):
    hi = x.astype(jnp.bfloat16).astype(jnp.float32)
    return hi, x - hi


def _fused_body(x_ref, vhi_ref, vlo_ref, pjhi_ref, pjlo_ref, gjhi_ref,
                gjlo_ref, phi_ref, plo_ref, tgt_col_ref, gts_bmat_ref,
                gts_col_ref, u_ref,
                iou_out_ref, cls_out_ref,
                maxprob_s, clsl_s, colsum_s):
    s = pl.program_id(0)
    nsteps = pl.num_programs(0)
    cc = x_ref.shape[1]
    m = vhi_ref.shape[0]
    c = vhi_ref.shape[1] - 1
    n_valid = m - (m // 100)

    def dot(x, y):
        return jax.lax.dot_general(x, y, (((1,), (0,)), ((), ())),
                                   preferred_element_type=jnp.float32)

    maxprob_s[pl.ds(s, 1), :] = jnp.max(x_ref[...], axis=(-2, -1))

    base = s * cc
    ones_row = jnp.ones((1, m), jnp.float32)
    lane_f = jax.lax.broadcasted_iota(jnp.int32, (cc, m), 1).astype(jnp.float32)
    pj_b = (dot(pjhi_ref[pl.ds(base, cc), :], ones_row) +
            dot(pjlo_ref[pl.ds(base, cc), :], ones_row))
    p_onehot = (pj_b == lane_f).astype(jnp.float32)
    g = dot(p_onehot, vhi_ref[...]) + dot(p_onehot, vlo_ref[...])
    a = g[:, 0:1]
    logits = g[:, 1:]

    gj_b = (dot(gjhi_ref[pl.ds(base, cc), :], ones_row) +
            dot(gjlo_ref[pl.ds(base, cc), :], ones_row))
    t_onehot = (gj_b == lane_f).astype(jnp.float32)
    tid = dot(t_onehot, tgt_col_ref[...])
    cls = tid.astype(jnp.int32)

    mx = jnp.max(logits, axis=1, keepdims=True)
    lse = mx + jnp.log(jnp.sum(jnp.exp(logits - mx), axis=1, keepdims=True))
    lane_c = jax.lax.broadcasted_iota(jnp.int32, (cc, c), 1)
    picked = jnp.sum(jnp.where(lane_c == cls, logits, 0.0), axis=1,
                     keepdims=True)
    clsl_s[pl.ds(base, cc), :] = lse - picked

    a_hi, a_lo = _split_hl(a)
    a_b = dot(a_hi, ones_row) + dot(a_lo, ones_row)
    d = jnp.abs(a_b - gts_bmat_ref[...])
    f = jnp.where(d < _THETA, d * d * (1.0 / (2.0 * _THETA)), d - 0.5 * _THETA)
    lmask = ((jax.lax.broadcasted_iota(jnp.int32, (1, cc), 1) + base)
             < n_valid).astype(jnp.float32)
    f_hi, f_lo = _split_hl(f)
    contrib = dot(lmask, f_hi) + dot(lmask, f_lo)

    @pl.when(s == 0)
    def _():
        colsum_s[...] = contrib

    @pl.when(s > 0)
    def _():
        colsum_s[...] += contrib

    @pl.when(s == nsteps - 1)
    def _():
        mat = maxprob_s[...]
        mat_hi, mat_lo = _split_hl(mat)
        lane_s = jax.lax.broadcasted_iota(jnp.int32, (m, nsteps), 1)
        row_sel = (phi_ref[...] == lane_s).astype(jnp.float32)
        t1 = dot(row_sel, mat_hi) + dot(row_sel, mat_lo)
        lane_cc = jax.lax.broadcasted_iota(jnp.int32, (m, cc), 1)
        mp = jnp.sum(jnp.where(plo_ref[...] == lane_cc, t1, 0.0), axis=1,
                     keepdims=True)
        removed = (mp < _ENT_THR) & (u_ref[...] < _RM_THR)
        w = jnp.where(removed, 0.0,
                      jnp.where(gts_col_ref[...] < _POS_IOU, 1.0, 2.0))
        valid = jax.lax.broadcasted_iota(jnp.int32, (m, 1), 0) < n_valid
        w = jnp.where(valid, w, 0.0)
        wsum = jnp.sum(w) + 0.0001
        cls_loss = jnp.sum(clsl_s[...] * w) / wsum
        cs_hi, cs_lo = _split_hl(colsum_s[...])
        iou_num = dot(cs_hi, w) + dot(cs_lo, w)
        iou_out_ref[...] = iou_num / wsum
        cls_out_ref[...] = jnp.reshape(cls_loss, (1, 1))


@jax.jit
def kernel(cls_logits, iou_scores, map_ious, pred_mask_prob, target_ids,
           map_indices):
    bs, ch, c = cls_logits.shape
    ht, wd = pred_mask_prob.shape[2], pred_mask_prob.shape[3]
    rows = bs * ch
    cc = 100
    nc = ch // cc
    nsteps = bs * nc

    k = ch - _FG
    pad = rows - bs * k
    zpad_i = jnp.zeros((pad,), jnp.int32)
    zpad_f = jnp.zeros((pad,), jnp.float32)

    pj = map_indices[:, 0, _FG:].astype(jnp.int32)
    gj = map_indices[:, 1, _FG:].astype(jnp.int32)
    off = (jnp.arange(bs, dtype=jnp.int32) * ch)[:, None]
    pjg = jnp.concatenate([(pj + off).reshape(-1), zpad_i])
    gjg = jnp.concatenate([(gj + off).reshape(-1), zpad_i])

    def _hl_col(x_int):
        xf = x_int.astype(jnp.float32)[:, None]
        hi = jax.lax.optimization_barrier(
            xf.astype(jnp.bfloat16)).astype(jnp.float32)
        return hi, xf - hi

    pjhi, pjlo = _hl_col(pjg)
    gjhi, gjlo = _hl_col(gjg)
    phi = (pjg // cc)[:, None]
    plo = (pjg % cc)[:, None]
    vmat = jnp.concatenate(
        [iou_scores.reshape(rows, 1), cls_logits.reshape(rows, c)], axis=1)
    vhi = jax.lax.optimization_barrier(
        vmat.astype(jnp.bfloat16)).astype(jnp.float32)
    vlo = vmat - vhi
    tgt_col = target_ids.astype(jnp.float32).reshape(rows, 1)
    iou = map_ious[:, _FG:].astype(jnp.float32).reshape(-1)
    iou_p = jnp.concatenate([iou, zpad_f])
    gts_bmat = jnp.broadcast_to(iou_p.reshape(1, rows), (cc, rows))
    gts_col = iou_p.reshape(rows, 1)
    u = jax.random.uniform(jax.random.key(42), (bs, k), dtype=jnp.float32)
    u_col = jnp.concatenate([u.reshape(-1), zpad_f]).reshape(rows, 1)

    full = lambda shape: pl.BlockSpec(shape, lambda s: (0,) * len(shape))
    iou_loss, cls_loss = pl.pallas_call(
        _fused_body,
        grid=(nsteps,),
        in_specs=[
            pl.BlockSpec((1, cc, ht, wd), lambda s: (s // nc, s % nc, 0, 0)),
            full((rows, 1 + c)),
            full((rows, 1 + c)),
            full((rows, 1)),
            full((rows, 1)),
            full((rows, 1)),
            full((rows, 1)),
            full((rows, 1)),
            full((rows, 1)),
            full((rows, 1)),
            full((cc, rows)),
            full((rows, 1)),
            full((rows, 1)),
        ],
        out_specs=[full((1, 1)), full((1, 1))],
        out_shape=[jax.ShapeDtypeStruct((1, 1), jnp.float32),
                   jax.ShapeDtypeStruct((1, 1), jnp.float32)],
        scratch_shapes=[
            pltpu.VMEM((nsteps, cc), jnp.float32),
            pltpu.VMEM((rows, 1), jnp.float32),
            pltpu.VMEM((1, rows), jnp.float32),
        ],
        interpret=_INTERPRET,
    )(pred_mask_prob, vhi, vlo, pjhi, pjlo, gjhi, gjlo, phi, plo,
      tgt_col, gts_bmat, gts_col, u_col)
    return (iou_loss[0, 0], cls_loss[0, 0])


# speedup vs baseline: 1.7429x; 1.0594x over previous
import jax
import jax.numpy as jnp
from jax.experimental import pallas as pl
from jax.experimental.pallas import tpu as pltpu

_FG = 1
_POS_IOU = 0.2
_ENT_THR = 0.1
_RM_THR = 0.9
_THETA = 0.1

_INTERPRET = False


def _split_hl(x):
    hi = x.astype(jnp.bfloat16).astype(jnp.float32)
    return hi, x - hi


def _fused_body(x_ref, vhi_ref, vlo_ref, pjhi_ref, pjlo_ref, gjhi_ref,
                gjlo_ref, phi_ref, plo_ref, tgt_col_ref, gts_bmat_ref,
                gts_col_ref, u_ref,
                iou_out_ref, cls_out_ref,
                maxprob_s, clsl_s, colsum_s):
    s = pl.program_id(0)
    nsteps = pl.num_programs(0)
    bb = x_ref.shape[0]
    ch = x_ref.shape[1]
    cc = bb * ch
    m = vhi_ref.shape[0]
    c = vhi_ref.shape[1] - 1
    n_valid = m - (m // 100)

    def dot(x, y):
        return jax.lax.dot_general(x, y, (((1,), (0,)), ((), ())),
                                   preferred_element_type=jnp.float32)

    maxprob_s[pl.ds(s * bb, bb), :] = jnp.max(x_ref[...], axis=(-2, -1))

    base = s * cc
    ones_row = jnp.ones((1, m), jnp.float32)
    lane_f = jax.lax.broadcasted_iota(jnp.int32, (cc, m), 1).astype(jnp.float32)
    pj_b = (dot(pjhi_ref[pl.ds(base, cc), :], ones_row) +
            dot(pjlo_ref[pl.ds(base, cc), :], ones_row))
    p_onehot = (pj_b == lane_f).astype(jnp.float32)
    g = dot(p_onehot, vhi_ref[...]) + dot(p_onehot, vlo_ref[...])
    a = g[:, 0:1]
    logits = g[:, 1:]

    gj_b = (dot(gjhi_ref[pl.ds(base, cc), :], ones_row) +
            dot(gjlo_ref[pl.ds(base, cc), :], ones_row))
    t_onehot = (gj_b == lane_f).astype(jnp.float32)
    tid = dot(t_onehot, tgt_col_ref[...])
    cls = tid.astype(jnp.int32)

    mx = jnp.max(logits, axis=1, keepdims=True)
    lse = mx + jnp.log(jnp.sum(jnp.exp(logits - mx), axis=1, keepdims=True))
    lane_c = jax.lax.broadcasted_iota(jnp.int32, (cc, c), 1)
    picked = jnp.sum(jnp.where(lane_c == cls, logits, 0.0), axis=1,
                     keepdims=True)
    clsl_s[pl.ds(base, cc), :] = lse - picked

    a_hi, a_lo = _split_hl(a)
    a_b = dot(a_hi, ones_row) + dot(a_lo, ones_row)
    d = jnp.abs(a_b - gts_bmat_ref[...])
    f = jnp.where(d < _THETA, d * d * (1.0 / (2.0 * _THETA)), d - 0.5 * _THETA)
    lmask = ((jax.lax.broadcasted_iota(jnp.int32, (1, cc), 1) + base)
             < n_valid).astype(jnp.float32)
    f_hi, f_lo = _split_hl(f)
    contrib = dot(lmask, f_hi) + dot(lmask, f_lo)

    @pl.when(s == 0)
    def _():
        colsum_s[...] = contrib

    @pl.when(s > 0)
    def _():
        colsum_s[...] += contrib

    @pl.when(s == nsteps - 1)
    def _():
        mat = maxprob_s[...]
        mat_hi, mat_lo = _split_hl(mat)
        lane_s = jax.lax.broadcasted_iota(jnp.int32, (m, nsteps * bb), 1)
        row_sel = (phi_ref[...] == lane_s).astype(jnp.float32)
        t1 = dot(row_sel, mat_hi) + dot(row_sel, mat_lo)
        lane_cc = jax.lax.broadcasted_iota(jnp.int32, (m, ch), 1)
        mp = jnp.sum(jnp.where(plo_ref[...] == lane_cc, t1, 0.0), axis=1,
                     keepdims=True)
        removed = (mp < _ENT_THR) & (u_ref[...] < _RM_THR)
        w = jnp.where(removed, 0.0,
                      jnp.where(gts_col_ref[...] < _POS_IOU, 1.0, 2.0))
        valid = jax.lax.broadcasted_iota(jnp.int32, (m, 1), 0) < n_valid
        w = jnp.where(valid, w, 0.0)
        wsum = jnp.sum(w) + 0.0001
        cls_loss = jnp.sum(clsl_s[...] * w) / wsum
        cs_hi, cs_lo = _split_hl(colsum_s[...])
        iou_num = dot(cs_hi, w) + dot(cs_lo, w)
        iou_out_ref[...] = iou_num / wsum
        cls_out_ref[...] = jnp.reshape(cls_loss, (1, 1))


@jax.jit
def kernel(cls_logits, iou_scores, map_ious, pred_mask_prob, target_ids,
           map_indices):
    bs, ch, c = cls_logits.shape
    ht, wd = pred_mask_prob.shape[2], pred_mask_prob.shape[3]
    rows = bs * ch
    bb = 2
    cc = bb * ch
    nsteps = bs // bb

    k = ch - _FG
    pad = rows - bs * k
    zpad_i = jnp.zeros((pad,), jnp.int32)
    zpad_f = jnp.zeros((pad,), jnp.float32)

    pj = map_indices[:, 0, _FG:].astype(jnp.int32)
    gj = map_indices[:, 1, _FG:].astype(jnp.int32)
    off = (jnp.arange(bs, dtype=jnp.int32) * ch)[:, None]
    pjg = jnp.concatenate([(pj + off).reshape(-1), zpad_i])
    gjg = jnp.concatenate([(gj + off).reshape(-1), zpad_i])

    def _hl_col(x_int):
        xf = x_int.astype(jnp.float32)[:, None]
        hi = jax.lax.optimization_barrier(
            xf.astype(jnp.bfloat16)).astype(jnp.float32)
        return hi, xf - hi

    pjhi, pjlo = _hl_col(pjg)
    gjhi, gjlo = _hl_col(gjg)
    phi = (pjg // ch)[:, None]
    plo = (pjg % ch)[:, None]
    vmat = jnp.concatenate(
        [iou_scores.reshape(rows, 1), cls_logits.reshape(rows, c)], axis=1)
    vhi = jax.lax.optimization_barrier(
        vmat.astype(jnp.bfloat16)).astype(jnp.float32)
    vlo = vmat - vhi
    tgt_col = target_ids.astype(jnp.float32).reshape(rows, 1)
    iou = map_ious[:, _FG:].astype(jnp.float32).reshape(-1)
    iou_p = jnp.concatenate([iou, zpad_f])
    gts_bmat = jnp.broadcast_to(iou_p.reshape(1, rows), (cc, rows))
    gts_col = iou_p.reshape(rows, 1)
    u = jax.random.uniform(jax.random.key(42), (bs, k), dtype=jnp.float32)
    u_col = jnp.concatenate([u.reshape(-1), zpad_f]).reshape(rows, 1)

    full = lambda shape: pl.BlockSpec(shape, lambda s: (0,) * len(shape))
    iou_loss, cls_loss = pl.pallas_call(
        _fused_body,
        grid=(nsteps,),
        in_specs=[
            pl.BlockSpec((bb, ch, ht, wd), lambda s: (s, 0, 0, 0)),
            full((rows, 1 + c)),
            full((rows, 1 + c)),
            full((rows, 1)),
            full((rows, 1)),
            full((rows, 1)),
            full((rows, 1)),
            full((rows, 1)),
            full((rows, 1)),
            full((rows, 1)),
            full((cc, rows)),
            full((rows, 1)),
            full((rows, 1)),
        ],
        out_specs=[full((1, 1)), full((1, 1))],
        out_shape=[jax.ShapeDtypeStruct((1, 1), jnp.float32),
                   jax.ShapeDtypeStruct((1, 1), jnp.float32)],
        scratch_shapes=[
            pltpu.VMEM((bs, ch), jnp.float32),
            pltpu.VMEM((rows, 1), jnp.float32),
            pltpu.VMEM((1, rows), jnp.float32),
        ],
        interpret=_INTERPRET,
    )(pred_mask_prob, vhi, vlo, pjhi, pjlo, gjhi, gjlo, phi, plo,
      tgt_col, gts_bmat, gts_col, u_col)
    return (iou_loss[0, 0], cls_loss[0, 0])


# speedup vs baseline: 1.9112x; 1.0965x over previous
import jax
import jax.numpy as jnp
from jax.experimental import pallas as pl
from jax.experimental.pallas import tpu as pltpu

_FG = 1
_POS_IOU = 0.2
_ENT_THR = 0.1
_RM_THR = 0.9
_THETA = 0.1

_INTERPRET = False


def _split_hl(x):
    hi = x.astype(jnp.bfloat16).astype(jnp.float32)
    return hi, x - hi


def _fused_body(x_ref, vhi_ref, vlo_ref, cols_ref, gts_hi_ref, gts_lo_ref,
                iou_out_ref, cls_out_ref,
                maxprob_s, clsl_s, colsum_s):
    s = pl.program_id(0)
    nsteps = pl.num_programs(0)
    bb = x_ref.shape[0]
    ch = x_ref.shape[1]
    cc = bb * ch
    m = vhi_ref.shape[0]
    c = vhi_ref.shape[1] - 1
    n_valid = m - (m // 100)

    def dot(x, y):
        return jax.lax.dot_general(x, y, (((1,), (0,)), ((), ())),
                                   preferred_element_type=jnp.float32)

    maxprob_s[pl.ds(s * bb, bb), :] = jnp.max(x_ref[...], axis=(-2, -1))

    base = s * cc
    ones_row = jnp.ones((1, m), jnp.float32)
    lane_f = jax.lax.broadcasted_iota(jnp.int32, (cc, m), 1).astype(jnp.float32)
    ck = cols_ref[pl.ds(base, cc), :]
    pj_b = dot(ck[:, 0:1], ones_row) + dot(ck[:, 1:2], ones_row)
    p_onehot = (pj_b == lane_f).astype(jnp.float32)
    g = dot(p_onehot, vhi_ref[...]) + dot(p_onehot, vlo_ref[...])
    a = g[:, 0:1]
    logits = g[:, 1:]

    gj_b = dot(ck[:, 2:3], ones_row) + dot(ck[:, 3:4], ones_row)
    t_onehot = (gj_b == lane_f).astype(jnp.float32)
    tid = dot(t_onehot, cols_ref[:, 6:7])
    cls = tid.astype(jnp.int32)

    mx = jnp.max(logits, axis=1, keepdims=True)
    lse = mx + jnp.log(jnp.sum(jnp.exp(logits - mx), axis=1, keepdims=True))
    lane_c = jax.lax.broadcasted_iota(jnp.int32, (cc, c), 1)
    picked = jnp.sum(jnp.where(lane_c == cls, logits, 0.0), axis=1,
                     keepdims=True)
    clsl_s[pl.ds(base, cc), :] = lse - picked

    a_hi, a_lo = _split_hl(a)
    a_b = dot(a_hi, ones_row) + dot(a_lo, ones_row)
    ones_col = jnp.ones((cc, 1), jnp.float32)
    g_b = dot(ones_col, gts_hi_ref[...]) + dot(ones_col, gts_lo_ref[...])
    d = jnp.abs(a_b - g_b)
    f = jnp.where(d < _THETA, d * d * (1.0 / (2.0 * _THETA)), d - 0.5 * _THETA)
    lmask = ((jax.lax.broadcasted_iota(jnp.int32, (1, cc), 1) + base)
             < n_valid).astype(jnp.float32)
    f_hi, f_lo = _split_hl(f)
    contrib = dot(lmask, f_hi) + dot(lmask, f_lo)

    @pl.when(s == 0)
    def _():
        colsum_s[...] = contrib

    @pl.when(s > 0)
    def _():
        colsum_s[...] += contrib

    @pl.when(s == nsteps - 1)
    def _():
        mat = maxprob_s[...]
        mat_hi, mat_lo = _split_hl(mat)
        lane_s = jax.lax.broadcasted_iota(
            jnp.int32, (m, nsteps * bb), 1).astype(jnp.float32)
        row_sel = (cols_ref[:, 4:5] == lane_s).astype(jnp.float32)
        t1 = dot(row_sel, mat_hi) + dot(row_sel, mat_lo)
        lane_cc = jax.lax.broadcasted_iota(
            jnp.int32, (m, ch), 1).astype(jnp.float32)
        mp = jnp.sum(jnp.where(cols_ref[:, 5:6] == lane_cc, t1, 0.0), axis=1,
                     keepdims=True)
        removed = (mp < _ENT_THR) & (cols_ref[:, 8:9] < _RM_THR)
        w = jnp.where(removed, 0.0,
                      jnp.where(cols_ref[:, 7:8] < _POS_IOU, 1.0, 2.0))
        valid = jax.lax.broadcasted_iota(jnp.int32, (m, 1), 0) < n_valid
        w = jnp.where(valid, w, 0.0)
        wsum = jnp.sum(w) + 0.0001
        cls_loss = jnp.sum(clsl_s[...] * w) / wsum
        cs_hi, cs_lo = _split_hl(colsum_s[...])
        iou_num = dot(cs_hi, w) + dot(cs_lo, w)
        iou_out_ref[...] = iou_num / wsum
        cls_out_ref[...] = jnp.reshape(cls_loss, (1, 1))


@jax.jit
def kernel(cls_logits, iou_scores, map_ious, pred_mask_prob, target_ids,
           map_indices):
    bs, ch, c = cls_logits.shape
    ht, wd = pred_mask_prob.shape[2], pred_mask_prob.shape[3]
    rows = bs * ch
    bb = 4
    cc = bb * ch
    nsteps = bs // bb

    k = ch - _FG
    pad = rows - bs * k
    zpad_i = jnp.zeros((pad,), jnp.int32)
    zpad_f = jnp.zeros((pad,), jnp.float32)

    pj = map_indices[:, 0, _FG:].astype(jnp.int32)
    gj = map_indices[:, 1, _FG:].astype(jnp.int32)
    off = (jnp.arange(bs, dtype=jnp.int32) * ch)[:, None]
    pjg = jnp.concatenate([(pj + off).reshape(-1), zpad_i])
    gjg = jnp.concatenate([(gj + off).reshape(-1), zpad_i])

    def _hl_col(x_int):
        xf = x_int.astype(jnp.float32)[:, None]
        hi = jax.lax.optimization_barrier(
            xf.astype(jnp.bfloat16)).astype(jnp.float32)
        return hi, xf - hi

    pjhi, pjlo = _hl_col(pjg)
    gjhi, gjlo = _hl_col(gjg)
    phi = (pjg // ch)[:, None].astype(jnp.float32)
    plo = (pjg % ch)[:, None].astype(jnp.float32)
    vmat = jnp.concatenate(
        [iou_scores.reshape(rows, 1), cls_logits.reshape(rows, c)], axis=1)
    vhi = jax.lax.optimization_barrier(
        vmat.astype(jnp.bfloat16)).astype(jnp.float32)
    vlo = vmat - vhi
    tgt_col = target_ids.astype(jnp.float32).reshape(rows, 1)
    iou = map_ious[:, _FG:].astype(jnp.float32).reshape(-1)
    iou_p = jnp.concatenate([iou, zpad_f])
    gts_row = iou_p.reshape(1, rows)
    gts_hi = jax.lax.optimization_barrier(
        gts_row.astype(jnp.bfloat16)).astype(jnp.float32)
    gts_lo = gts_row - gts_hi
    gts_col = iou_p.reshape(rows, 1)
    u = jax.random.uniform(jax.random.key(42), (bs, k), dtype=jnp.float32)
    u_col = jnp.concatenate([u.reshape(-1), zpad_f]).reshape(rows, 1)
    cols = jnp.concatenate([pjhi, pjlo, gjhi, gjlo, phi, plo,
                            tgt_col, gts_col, u_col], axis=1)

    full = lambda shape: pl.BlockSpec(shape, lambda s: (0,) * len(shape))
    iou_loss, cls_loss = pl.pallas_call(
        _fused_body,
        grid=(nsteps,),
        in_specs=[
            pl.BlockSpec((bb, ch, ht, wd), lambda s: (s, 0, 0, 0)),
            full((rows, 1 + c)),
            full((rows, 1 + c)),
            full((rows, 9)),
            full((1, rows)),
            full((1, rows)),
        ],
        out_specs=[full((1, 1)), full((1, 1))],
        out_shape=[jax.ShapeDtypeStruct((1, 1), jnp.float32),
                   jax.ShapeDtypeStruct((1, 1), jnp.float32)],
        scratch_shapes=[
            pltpu.VMEM((bs, ch), jnp.float32),
            pltpu.VMEM((rows, 1), jnp.float32),
            pltpu.VMEM((1, rows), jnp.float32),
        ],
        compiler_params=pltpu.CompilerParams(
            vmem_limit_bytes=100 * 1024 * 1024),
        interpret=_INTERPRET,
    )(pred_mask_prob, vhi, vlo, cols, gts_hi, gts_lo)
    return (iou_loss[0, 0], cls_loss[0, 0])


# speedup vs baseline: 1.9219x; 1.0056x over previous
import jax
import jax.numpy as jnp
from jax.experimental import pallas as pl
from jax.experimental.pallas import tpu as pltpu

_FG = 1
_POS_IOU = 0.2
_ENT_THR = 0.1
_RM_THR = 0.9
_THETA = 0.1

def _split_hl(x):
    hi = x.astype(jnp.bfloat16).astype(jnp.float32)
    return hi, x - hi


def _fused_body(x_ref, vhi_ref, vlo_ref, cols_ref, gts_hi_ref, gts_lo_ref,
                iou_out_ref, cls_out_ref,
                maxprob_s, clsl_s, colsum_s):
    s = pl.program_id(0)
    nsteps = pl.num_programs(0)
    bb = x_ref.shape[0]
    ch = x_ref.shape[1]
    cc = bb * ch
    m = vhi_ref.shape[0]
    c = vhi_ref.shape[1] - 1
    n_valid = m - (m // 100)

    def dot(x, y):
        return jax.lax.dot_general(x, y, (((1,), (0,)), ((), ())),
                                   preferred_element_type=jnp.float32)

    maxprob_s[pl.ds(s * bb, bb), :] = jnp.max(x_ref[...], axis=(-2, -1))

    base = s * cc
    ones_row = jnp.ones((1, m), jnp.float32)
    lane_f = jax.lax.broadcasted_iota(jnp.int32, (cc, m), 1).astype(jnp.float32)
    ck = cols_ref[pl.ds(base, cc), :]
    pj_b = dot(ck[:, 0:1], ones_row) + dot(ck[:, 1:2], ones_row)
    p_onehot = (pj_b == lane_f).astype(jnp.float32)
    g = dot(p_onehot, vhi_ref[...]) + dot(p_onehot, vlo_ref[...])
    a = g[:, 0:1]
    logits = g[:, 1:]

    gj_b = dot(ck[:, 2:3], ones_row) + dot(ck[:, 3:4], ones_row)
    t_onehot = (gj_b == lane_f).astype(jnp.float32)
    tid = dot(t_onehot, cols_ref[:, 6:7])
    cls = tid.astype(jnp.int32)

    mx = jnp.max(logits, axis=1, keepdims=True)
    lse = mx + jnp.log(jnp.sum(jnp.exp(logits - mx), axis=1, keepdims=True))
    lane_c = jax.lax.broadcasted_iota(jnp.int32, (cc, c), 1)
    picked = jnp.sum(jnp.where(lane_c == cls, logits, 0.0), axis=1,
                     keepdims=True)
    clsl_s[pl.ds(base, cc), :] = lse - picked

    a_hi, a_lo = _split_hl(a)
    a_b = dot(a_hi, ones_row) + dot(a_lo, ones_row)
    ones_col = jnp.ones((cc, 1), jnp.float32)
    g_b = dot(ones_col, gts_hi_ref[...]) + dot(ones_col, gts_lo_ref[...])
    d = jnp.abs(a_b - g_b)
    f = jnp.where(d < _THETA, d * d * (1.0 / (2.0 * _THETA)), d - 0.5 * _THETA)
    lmask = ((jax.lax.broadcasted_iota(jnp.int32, (1, cc), 1) + base)
             < n_valid).astype(jnp.float32)
    f_hi, f_lo = _split_hl(f)
    contrib = dot(lmask, f_hi) + dot(lmask, f_lo)

    @pl.when(s == 0)
    def _():
        colsum_s[...] = contrib

    @pl.when(s > 0)
    def _():
        colsum_s[...] += contrib

    @pl.when(s == nsteps - 1)
    def _():
        mat = maxprob_s[...]
        mat_hi, mat_lo = _split_hl(mat)
        lane_s = jax.lax.broadcasted_iota(
            jnp.int32, (m, nsteps * bb), 1).astype(jnp.float32)
        row_sel = (cols_ref[:, 4:5] == lane_s).astype(jnp.float32)
        t1 = dot(row_sel, mat_hi) + dot(row_sel, mat_lo)
        lane_cc = jax.lax.broadcasted_iota(
            jnp.int32, (m, ch), 1).astype(jnp.float32)
        mp = jnp.sum(jnp.where(cols_ref[:, 5:6] == lane_cc, t1, 0.0), axis=1,
                     keepdims=True)
        removed = (mp < _ENT_THR) & (cols_ref[:, 8:9] < _RM_THR)
        w = jnp.where(removed, 0.0,
                      jnp.where(cols_ref[:, 7:8] < _POS_IOU, 1.0, 2.0))
        valid = jax.lax.broadcasted_iota(jnp.int32, (m, 1), 0) < n_valid
        w = jnp.where(valid, w, 0.0)
        wsum = jnp.sum(w) + 0.0001
        cls_loss = jnp.sum(clsl_s[...] * w) / wsum
        cs_hi, cs_lo = _split_hl(colsum_s[...])
        iou_num = dot(cs_hi, w) + dot(cs_lo, w)
        iou_out_ref[...] = iou_num / wsum
        cls_out_ref[...] = jnp.reshape(cls_loss, (1, 1))


@jax.jit
def kernel(cls_logits, iou_scores, map_ious, pred_mask_prob, target_ids,
           map_indices):
    bs, ch, c = cls_logits.shape
    ht, wd = pred_mask_prob.shape[2], pred_mask_prob.shape[3]
    rows = bs * ch
    bb = 4
    cc = bb * ch
    nsteps = bs // bb

    k = ch - _FG
    pad = rows - bs * k
    zpad_i = jnp.zeros((pad,), jnp.int32)
    zpad_f = jnp.zeros((pad,), jnp.float32)

    pj = map_indices[:, 0, _FG:].astype(jnp.int32)
    gj = map_indices[:, 1, _FG:].astype(jnp.int32)
    off = (jnp.arange(bs, dtype=jnp.int32) * ch)[:, None]
    pjg = jnp.concatenate([(pj + off).reshape(-1), zpad_i])
    gjg = jnp.concatenate([(gj + off).reshape(-1), zpad_i])

    def _hl_col(x_int):
        xf = x_int.astype(jnp.float32)[:, None]
        hi = jax.lax.optimization_barrier(
            xf.astype(jnp.bfloat16)).astype(jnp.float32)
        return hi, xf - hi

    pjhi, pjlo = _hl_col(pjg)
    gjhi, gjlo = _hl_col(gjg)
    phi = (pjg // ch)[:, None].astype(jnp.float32)
    plo = (pjg % ch)[:, None].astype(jnp.float32)
    vmat = jnp.concatenate(
        [iou_scores.reshape(rows, 1), cls_logits.reshape(rows, c)], axis=1)
    vhi = jax.lax.optimization_barrier(
        vmat.astype(jnp.bfloat16)).astype(jnp.float32)
    vlo = vmat - vhi
    tgt_col = target_ids.astype(jnp.float32).reshape(rows, 1)
    iou = map_ious[:, _FG:].astype(jnp.float32).reshape(-1)
    iou_p = jnp.concatenate([iou, zpad_f])
    gts_row = iou_p.reshape(1, rows)
    gts_hi = jax.lax.optimization_barrier(
        gts_row.astype(jnp.bfloat16)).astype(jnp.float32)
    gts_lo = gts_row - gts_hi
    gts_col = iou_p.reshape(rows, 1)
    u = jax.random.uniform(jax.random.key(42), (bs, k), dtype=jnp.float32)
    u_col = jnp.concatenate([u.reshape(-1), zpad_f]).reshape(rows, 1)
    cols = jnp.concatenate([pjhi, pjlo, gjhi, gjlo, phi, plo,
                            tgt_col, gts_col, u_col], axis=1)

    full = lambda shape: pl.BlockSpec(shape, lambda s: (0,) * len(shape))
    iou_loss, cls_loss = pl.pallas_call(
        _fused_body,
        grid=(nsteps,),
        in_specs=[
            pl.BlockSpec((bb, ch, ht, wd), lambda s: (s, 0, 0, 0)),
            full((rows, 1 + c)),
            full((rows, 1 + c)),
            full((rows, 9)),
            full((1, rows)),
            full((1, rows)),
        ],
        out_specs=[full((1, 1)), full((1, 1))],
        out_shape=[jax.ShapeDtypeStruct((1, 1), jnp.float32),
                   jax.ShapeDtypeStruct((1, 1), jnp.float32)],
        scratch_shapes=[
            pltpu.VMEM((bs, ch), jnp.float32),
            pltpu.VMEM((rows, 1), jnp.float32),
            pltpu.VMEM((1, rows), jnp.float32),
        ],
        compiler_params=pltpu.CompilerParams(
            vmem_limit_bytes=100 * 1024 * 1024),
    )(pred_mask_prob, vhi, vlo, cols, gts_hi, gts_lo)
    return (iou_loss[0, 0], cls_loss[0, 0])
